# baseline (speedup 1.0000x reference)
import functools

import jax
import jax.numpy as jnp
import numpy as np
from jax import lax
from jax.experimental import pallas as pl
from jax.experimental.pallas import tpu as pltpu
from jax.experimental.pallas import tpu_sc as plsc

B = 4096
L = 200
LP = 208
D = 64
NLAB = 500
NLAB_PAD = 512

NC = 2
NS = 16
NW = NC * NS
RPW = B // NW

SPLITS = ((0, 104), (104, 96))

PERM = np.concatenate([
    np.arange(0, 32, 2), np.arange(1, 32, 2),
    np.arange(32, 64, 2), np.arange(33, 64, 2),
])


def _sc_pool_body(x_hbm, table_hbm, out_hbm, idx_v, rows_v, out_v, t0_v,
                  sem0, sem1):
    wid = lax.axis_index("s") * NC + lax.axis_index("c")
    base = wid * RPW
    sems = (sem0, sem1)

    pltpu.sync_copy(x_hbm.at[pl.ds(base, RPW)], idx_v)
    pltpu.sync_copy(table_hbm.at[0], t0_v)

    def fire(r, buf):
        for off, n in SPLITS:
            pltpu.async_copy(
                table_hbm.at[idx_v.at[r, pl.ds(off, n)]],
                rows_v.at[buf, pl.ds(off, n)],
                sems[buf])

    def drain(buf):
        for off, n in SPLITS:
            pltpu.make_async_copy(
                table_hbm.at[pl.ds(0, n)],
                rows_v.at[buf, pl.ds(off, n)],
                sems[buf]).wait()

    lane = lax.iota(jnp.int32, 16)

    t0 = []
    for half in range(2):
        e, o = plsc.unpack(t0_v[pl.ds(half * 32, 32)],
                           format=plsc.PackFormat.INTERLEAVED,
                           preferred_element_type=jnp.float32)
        t0 += [e, o]

    def compute(r, buf):
        def inner(j, acc):
            new = []
            for half in range(2):
                v = rows_v[buf, j, pl.ds(half * 32, 32)]
                e, o = plsc.unpack(v, format=plsc.PackFormat.INTERLEAVED,
                                   preferred_element_type=jnp.float32)
                new += [acc[2 * half] + e, acc[2 * half + 1] + o]
            return tuple(new)
        acc = lax.fori_loop(
            0, L, inner,
            tuple(jnp.zeros((16,), jnp.float32) for _ in range(4)),
            unroll=4)
        cnt = jnp.zeros((16,), jnp.int32)
        for t in range(LP // 16 - 1):
            z = idx_v[r, pl.ds(t * 16, 16)] == 0
            cnt = cnt + plsc.all_reduce_population_count(z)
        ztail = (idx_v[r, pl.ds(LP - 16, 16)] == 0) & (lane < 16 - (LP - L))
        cnt = cnt + plsc.all_reduce_population_count(ztail)
        cf = cnt.astype(jnp.float32)
        for k in range(4):
            out_v[r, pl.ds(k * 16, 16)] = acc[k] - cf * t0[k]

    fire(0, 0)

    def pair(i, carry):
        r0 = 2 * i
        fire(r0 + 1, 1)
        drain(0)
        compute(r0, 0)

        @pl.when(r0 + 2 < RPW)
        def _():
            fire(r0 + 2, 0)

        drain(1)
        compute(r0 + 1, 1)
        return carry

    lax.fori_loop(0, RPW // 2, pair, 0)
    pltpu.sync_copy(out_v, out_hbm.at[pl.ds(base, RPW)])


_sc_pool = pl.kernel(
    _sc_pool_body,
    out_type=jax.ShapeDtypeStruct((B, D), jnp.float32),
    mesh=plsc.VectorSubcoreMesh(
        core_axis_name="c", subcore_axis_name="s",
        num_cores=NC, num_subcores=NS),
    scratch_types=[
        pltpu.VMEM((RPW, LP), jnp.int32),
        pltpu.VMEM((2, L, D), jnp.bfloat16),
        pltpu.VMEM((RPW, D), jnp.float32),
        pltpu.VMEM((D,), jnp.bfloat16),
        pltpu.SemaphoreType.DMA,
        pltpu.SemaphoreType.DMA,
    ],
    compiler_params=pltpu.CompilerParams(
        use_tc_tiling_on_sc=False, needs_layout_passes=False),
)


def _tc_head_body(p_ref, w_ref, b_ref, o_ref):
    h = jnp.tanh(p_ref[...])
    logits = jnp.dot(h, w_ref[...],
                     preferred_element_type=jnp.float32) + b_ref[...]
    m = jnp.max(logits, axis=-1, keepdims=True)
    e = jnp.exp(logits - m)
    o_ref[...] = e / jnp.sum(e, axis=-1, keepdims=True)


TB = 256


@functools.partial(jax.jit, static_argnums=())
def kernel(x, table, W, b):
    xp = jnp.pad(x.astype(jnp.int32), ((0, 0), (0, LP - L)),
                 constant_values=1)
    pooled = _sc_pool(xp, table.astype(jnp.bfloat16))
    Wp = jnp.pad(W[PERM], ((0, 0), (0, NLAB_PAD - NLAB)))
    bp = jnp.concatenate(
        [b, jnp.full((NLAB_PAD - NLAB,), -1e30, b.dtype)]).reshape(1, NLAB_PAD)
    out = pl.pallas_call(
        _tc_head_body,
        grid=(B // TB,),
        in_specs=[
            pl.BlockSpec((TB, D), lambda i: (i, 0)),
            pl.BlockSpec((D, NLAB_PAD), lambda i: (0, 0)),
            pl.BlockSpec((1, NLAB_PAD), lambda i: (0, 0)),
        ],
        out_specs=pl.BlockSpec((TB, NLAB_PAD), lambda i: (i, 0)),
        out_shape=jax.ShapeDtypeStruct((B, NLAB_PAD), jnp.float32),
    )(pooled, Wp, bp)
    return out[:, :NLAB]


# speedup vs baseline: 1.0893x; 1.0893x over previous
import functools

import jax
import jax.numpy as jnp
from jax import lax
from jax.experimental import pallas as pl
from jax.experimental.pallas import tpu as pltpu
from jax.experimental.pallas import tpu_sc as plsc

B = 4096
L = 200
D = 64
V = 100001
NLAB = 500
NLAB_PAD = 512

NC = 2
NS = 16
NW = NC * NS
RPW = B // NW

SPLITS = ((0, 104), (104, 96))

CAST_CH = 625
CAST_CHUNKS = 5

_SC_PARAMS = pltpu.CompilerParams(
    use_tc_tiling_on_sc=False, needs_layout_passes=False)
_MESH = plsc.VectorSubcoreMesh(
    core_axis_name="c", subcore_axis_name="s",
    num_cores=NC, num_subcores=NS)


def _cast_chunk(in2, out2, nrows):
    def body(row, carry):
        for half in range(2):
            a = in2[row, pl.ds(half * 32, 16)]
            b = in2[row, pl.ds(half * 32 + 16, 16)]
            out2[row, pl.ds(half * 32, 32)] = plsc.pack(
                a, b, format=plsc.PackFormat.INTERLEAVED)
        return carry
    lax.fori_loop(0, nrows, body, 0, unroll=4)


def _sc_cast_body(tf_hbm, tb_hbm, in_v, out_v, sem0, sem1):
    wid = lax.axis_index("s") * NC + lax.axis_index("c")
    base = wid * (CAST_CH * CAST_CHUNKS)
    sems = (sem0, sem1)
    ins = (in_v.at[0], in_v.at[1])
    outs = (out_v.at[0], out_v.at[1])

    def fire(c, buf):
        pltpu.async_copy(tf_hbm.at[pl.ds(base + c * CAST_CH, CAST_CH)],
                         ins[buf], sems[buf])

    def drain(buf):
        pltpu.make_async_copy(tf_hbm.at[pl.ds(0, CAST_CH)],
                              ins[buf], sems[buf]).wait()

    fire(0, 0)
    for c in range(CAST_CHUNKS):
        buf = c % 2
        if c + 1 < CAST_CHUNKS:
            fire(c + 1, 1 - buf)
        drain(buf)
        _cast_chunk(ins[buf], outs[buf], CAST_CH)
        pltpu.sync_copy(outs[buf],
                        tb_hbm.at[pl.ds(base + c * CAST_CH, CAST_CH)])

    @pl.when(wid == NW - 1)
    def _():
        pltpu.sync_copy(tf_hbm.at[pl.ds(V - 1, 1)], in_v.at[0, pl.ds(0, 1)])
        _cast_chunk(in_v.at[0], out_v.at[0], 1)
        pltpu.sync_copy(out_v.at[0, pl.ds(0, 1)],
                        tb_hbm.at[pl.ds(V - 1, 1)])


_sc_cast = pl.kernel(
    _sc_cast_body,
    out_type=jax.ShapeDtypeStruct((V, D), jnp.bfloat16),
    mesh=_MESH,
    scratch_types=[
        pltpu.VMEM((2, CAST_CH, D), jnp.float32),
        pltpu.VMEM((2, CAST_CH, D), jnp.bfloat16),
        pltpu.SemaphoreType.DMA,
        pltpu.SemaphoreType.DMA,
    ],
    compiler_params=_SC_PARAMS,
)


def _sc_pool_body(x_hbm, table_hbm, out_hbm, idx_v, rows_v, out_v, t0_v,
                  sem0, sem1):
    wid = lax.axis_index("s") * NC + lax.axis_index("c")
    base = wid * RPW
    sems = (sem0, sem1)

    pltpu.sync_copy(x_hbm.at[pl.ds(base, RPW)], idx_v)
    pltpu.sync_copy(table_hbm.at[0], t0_v)

    def fire(r, buf):
        for off, n in SPLITS:
            pltpu.async_copy(
                table_hbm.at[idx_v.at[r, pl.ds(off, n)]],
                rows_v.at[buf, pl.ds(off, n)],
                sems[buf])

    def drain(buf):
        for off, n in SPLITS:
            pltpu.make_async_copy(
                table_hbm.at[pl.ds(0, n)],
                rows_v.at[buf, pl.ds(off, n)],
                sems[buf]).wait()

    lane = lax.iota(jnp.int32, 16)

    t0 = []
    for half in range(2):
        e, o = plsc.unpack(t0_v[pl.ds(half * 32, 32)],
                           format=plsc.PackFormat.INTERLEAVED,
                           preferred_element_type=jnp.float32)
        t0 += [e, o]

    def compute(r, buf):
        def inner(j, acc):
            new = []
            for half in range(2):
                v = rows_v[buf, j, pl.ds(half * 32, 32)]
                e, o = plsc.unpack(v, format=plsc.PackFormat.INTERLEAVED,
                                   preferred_element_type=jnp.float32)
                new += [acc[2 * half] + e, acc[2 * half + 1] + o]
            return tuple(new)
        acc = lax.fori_loop(
            0, L, inner,
            tuple(jnp.zeros((16,), jnp.float32) for _ in range(4)),
            unroll=4)
        cnt = jnp.zeros((16,), jnp.int32)
        for t in range(L // 16):
            z = idx_v[r, pl.ds(t * 16, 16)] == 0
            cnt = cnt + plsc.all_reduce_population_count(z)
        ztail = (idx_v[r, pl.ds(L - 16, 16)] == 0) & (lane >= 8)
        cnt = cnt + plsc.all_reduce_population_count(ztail)
        cf = cnt.astype(jnp.float32)
        for k in range(4):
            out_v[r, pl.ds(k * 16, 16)] = acc[k] - cf * t0[k]

    fire(0, 0)

    def pair(i, carry):
        r0 = 2 * i
        fire(r0 + 1, 1)
        drain(0)
        compute(r0, 0)

        @pl.when(r0 + 2 < RPW)
        def _():
            fire(r0 + 2, 0)

        drain(1)
        compute(r0 + 1, 1)
        return carry

    lax.fori_loop(0, RPW // 2, pair, 0)
    pltpu.sync_copy(out_v, out_hbm.at[pl.ds(base, RPW)])


_sc_pool = pl.kernel(
    _sc_pool_body,
    out_type=jax.ShapeDtypeStruct((B, D), jnp.float32),
    mesh=_MESH,
    scratch_types=[
        pltpu.VMEM((RPW, L), jnp.int32),
        pltpu.VMEM((2, L, D), jnp.bfloat16),
        pltpu.VMEM((RPW, D), jnp.float32),
        pltpu.VMEM((D,), jnp.bfloat16),
        pltpu.SemaphoreType.DMA,
        pltpu.SemaphoreType.DMA,
    ],
    compiler_params=_SC_PARAMS,
)


def _tc_head_body(p_ref, w_ref, b_ref, o_ref):
    h = jnp.tanh(p_ref[...])
    logits = jnp.dot(h, w_ref[...],
                     preferred_element_type=jnp.float32) + b_ref[...]
    m = jnp.max(logits, axis=-1, keepdims=True)
    e = jnp.exp(logits - m)
    o_ref[...] = e / jnp.sum(e, axis=-1, keepdims=True)


TB = 256


@functools.partial(jax.jit, static_argnums=())
def kernel(x, table, W, b):
    table_bf = _sc_cast(table)
    pooled = _sc_pool(x, table_bf)
    Wp = jnp.pad(W, ((0, 0), (0, NLAB_PAD - NLAB)))
    bp = jnp.concatenate(
        [b, jnp.full((NLAB_PAD - NLAB,), -1e30, b.dtype)]).reshape(1, NLAB_PAD)
    out = pl.pallas_call(
        _tc_head_body,
        grid=(B // TB,),
        in_specs=[
            pl.BlockSpec((TB, D), lambda i: (i, 0)),
            pl.BlockSpec((D, NLAB_PAD), lambda i: (0, 0)),
            pl.BlockSpec((1, NLAB_PAD), lambda i: (0, 0)),
        ],
        out_specs=pl.BlockSpec((TB, NLAB_PAD), lambda i: (i, 0)),
        out_shape=jax.ShapeDtypeStruct((B, NLAB_PAD), jnp.float32),
    )(pooled, Wp, bp)
    return out[:, :NLAB]


# speedup vs baseline: 1.1442x; 1.0503x over previous
import functools

import jax
import jax.numpy as jnp
from jax import lax
from jax.experimental import pallas as pl
from jax.experimental.pallas import tpu as pltpu
from jax.experimental.pallas import tpu_sc as plsc

B = 4096
L = 200
D = 64
V = 100001
NLAB = 500
NLAB_PAD = 512

NC = 2
NS = 16
NW = NC * NS
RPW = B // NW

SPLITS = ((0, 104), (104, 96))

V_ROWS = 100001

_SC_PARAMS = pltpu.CompilerParams(
    use_tc_tiling_on_sc=False, needs_layout_passes=False)
_MESH = plsc.VectorSubcoreMesh(
    core_axis_name="c", subcore_axis_name="s",
    num_cores=NC, num_subcores=NS)


def _sc_pool_body(x_hbm, table_hbm, out_hbm, idx_v, rows_v, out_v, t0_v,
                  sem0, sem1):
    wid = lax.axis_index("s") * NC + lax.axis_index("c")
    base = wid * RPW
    sems = (sem0, sem1)

    pltpu.sync_copy(x_hbm.at[pl.ds(base, RPW)], idx_v)
    pltpu.sync_copy(table_hbm.at[pl.ds(0, 1)], t0_v)

    def fire(r, buf):
        for off, n in SPLITS:
            pltpu.async_copy(
                table_hbm.at[idx_v.at[r, pl.ds(off, n)]],
                rows_v.at[buf, pl.ds(off, n)],
                sems[buf])

    def drain(buf):
        for off, n in SPLITS:
            pltpu.make_async_copy(
                table_hbm.at[pl.ds(0, n)],
                rows_v.at[buf, pl.ds(off, n)],
                sems[buf]).wait()

    lane = lax.iota(jnp.int32, 16)

    t0 = [t0_v[0, pl.ds(k * 16, 16)] for k in range(4)]

    def compute(r, buf):
        def inner(j, acc):
            return tuple(acc[k] + rows_v[buf, j, pl.ds(k * 16, 16)]
                         for k in range(4))
        acc = lax.fori_loop(
            0, L, inner,
            tuple(jnp.zeros((16,), jnp.float32) for _ in range(4)),
            unroll=4)
        cnt = jnp.zeros((16,), jnp.int32)
        for t in range(L // 16):
            z = idx_v[r, pl.ds(t * 16, 16)] == 0
            cnt = cnt + plsc.all_reduce_population_count(z)
        ztail = (idx_v[r, pl.ds(L - 16, 16)] == 0) & (lane >= 8)
        cnt = cnt + plsc.all_reduce_population_count(ztail)
        cf = cnt.astype(jnp.float32)
        for k in range(4):
            out_v[r, pl.ds(k * 16, 16)] = acc[k] - cf * t0[k]

    fire(0, 0)

    def pair(i, carry):
        r0 = 2 * i
        fire(r0 + 1, 1)
        drain(0)
        compute(r0, 0)

        @pl.when(r0 + 2 < RPW)
        def _():
            fire(r0 + 2, 0)

        drain(1)
        compute(r0 + 1, 1)
        return carry

    lax.fori_loop(0, RPW // 2, pair, 0)
    pltpu.sync_copy(out_v, out_hbm.at[pl.ds(base, RPW)])


_sc_pool = pl.kernel(
    _sc_pool_body,
    out_type=jax.ShapeDtypeStruct((B, D), jnp.float32),
    mesh=_MESH,
    scratch_types=[
        pltpu.VMEM((RPW, L), jnp.int32),
        pltpu.VMEM((2, L, D), jnp.float32),
        pltpu.VMEM((RPW, D), jnp.float32),
        pltpu.VMEM((1, D), jnp.float32),
        pltpu.SemaphoreType.DMA,
        pltpu.SemaphoreType.DMA,
    ],
    compiler_params=_SC_PARAMS,
)


def _tc_head_body(p_ref, w_ref, b_ref, o_ref):
    h = jnp.tanh(p_ref[...])
    logits = jnp.dot(h, w_ref[...],
                     preferred_element_type=jnp.float32) + b_ref[...]
    m = jnp.max(logits, axis=-1, keepdims=True)
    e = jnp.exp(logits - m)
    o_ref[...] = e / jnp.sum(e, axis=-1, keepdims=True)


TB = 256


@functools.partial(jax.jit, static_argnums=())
def kernel(x, table, W, b):
    pooled = _sc_pool(x, table)
    Wp = jnp.pad(W, ((0, 0), (0, NLAB_PAD - NLAB)))
    bp = jnp.concatenate(
        [b, jnp.full((NLAB_PAD - NLAB,), -1e30, b.dtype)]).reshape(1, NLAB_PAD)
    out = pl.pallas_call(
        _tc_head_body,
        grid=(B // TB,),
        in_specs=[
            pl.BlockSpec((TB, D), lambda i: (i, 0)),
            pl.BlockSpec((D, NLAB_PAD), lambda i: (0, 0)),
            pl.BlockSpec((1, NLAB_PAD), lambda i: (0, 0)),
        ],
        out_specs=pl.BlockSpec((TB, NLAB_PAD), lambda i: (i, 0)),
        out_shape=jax.ShapeDtypeStruct((B, NLAB_PAD), jnp.float32),
    )(pooled, Wp, bp)
    return out[:, :NLAB]


# speedup vs baseline: 1.1470x; 1.0025x over previous
import functools

import jax
import jax.numpy as jnp
from jax import lax
from jax.experimental import pallas as pl
from jax.experimental.pallas import tpu as pltpu
from jax.experimental.pallas import tpu_sc as plsc

B = 4096
L = 200
D = 64
V = 100001
NLAB = 500
NLAB_PAD = 512

NC = 2
NS = 16
NW = NC * NS
RPW = B // NW

SPLITS = ((0, 104), (104, 96))

V_ROWS = 100001

_SC_PARAMS = pltpu.CompilerParams(
    use_tc_tiling_on_sc=False, needs_layout_passes=False)
_MESH = plsc.VectorSubcoreMesh(
    core_axis_name="c", subcore_axis_name="s",
    num_cores=NC, num_subcores=NS)


def _sc_pool_body(x_hbm, table_hbm, out_hbm, idx_v, rows_v, out_v, t0_v,
                  sem0, sem1):
    wid = lax.axis_index("s") * NC + lax.axis_index("c")
    base = wid * RPW
    sems = (sem0, sem1)

    pltpu.sync_copy(x_hbm.at[pl.ds(base, RPW)], idx_v)
    pltpu.sync_copy(table_hbm.at[pl.ds(0, 1)], t0_v)

    def fire(r, buf):
        for off, n in SPLITS:
            pltpu.async_copy(
                table_hbm.at[idx_v.at[r, pl.ds(off, n)]],
                rows_v.at[buf, pl.ds(off, n)],
                sems[buf])

    def drain(buf):
        for off, n in SPLITS:
            pltpu.make_async_copy(
                table_hbm.at[pl.ds(0, n)],
                rows_v.at[buf, pl.ds(off, n)],
                sems[buf]).wait()

    lane = lax.iota(jnp.int32, 16)

    t0 = [t0_v[0, pl.ds(k * 16, 16)] for k in range(4)]

    def compute(r, buf):
        def inner(j, acc):
            return tuple(acc[k] + rows_v[buf, j, pl.ds(k * 16, 16)]
                         for k in range(4))
        acc = lax.fori_loop(
            0, L, inner,
            tuple(jnp.zeros((16,), jnp.float32) for _ in range(4)),
            unroll=4)
        cnt = jnp.zeros((16,), jnp.int32)
        for t in range(L // 16):
            z = idx_v[r, pl.ds(t * 16, 16)] == 0
            cnt = cnt + plsc.all_reduce_population_count(z)
        ztail = (idx_v[r, pl.ds(L - 16, 16)] == 0) & (lane >= 8)
        cnt = cnt + plsc.all_reduce_population_count(ztail)
        cf = cnt.astype(jnp.float32)
        for k in range(4):
            out_v[r, pl.ds(k * 16, 16)] = acc[k] - cf * t0[k]

    fire(0, 0)

    def pair(i, carry):
        r0 = 2 * i
        fire(r0 + 1, 1)
        drain(0)
        compute(r0, 0)

        @pl.when(r0 + 2 < RPW)
        def _():
            fire(r0 + 2, 0)

        drain(1)
        compute(r0 + 1, 1)
        return carry

    lax.fori_loop(0, RPW // 2, pair, 0)
    pltpu.sync_copy(out_v, out_hbm.at[pl.ds(base, RPW)])


_sc_pool = pl.kernel(
    _sc_pool_body,
    out_type=jax.ShapeDtypeStruct((B, D), jnp.float32),
    mesh=_MESH,
    scratch_types=[
        pltpu.VMEM((RPW, L), jnp.int32),
        pltpu.VMEM((2, L, D), jnp.float32),
        pltpu.VMEM((RPW, D), jnp.float32),
        pltpu.VMEM((1, D), jnp.float32),
        pltpu.SemaphoreType.DMA,
        pltpu.SemaphoreType.DMA,
    ],
    compiler_params=_SC_PARAMS,
)


def _tc_head_body(p_ref, w_ref, b_ref, o_ref):
    h = jnp.tanh(p_ref[...])
    logits = jnp.dot(h, w_ref[...],
                     preferred_element_type=jnp.float32) + b_ref[...]
    m = jnp.max(logits, axis=-1, keepdims=True)
    e = jnp.exp(logits - m)
    p = e / jnp.sum(e, axis=-1, keepdims=True)
    o_ref[...] = p[:, :NLAB]


TB = 256


@functools.partial(jax.jit, static_argnums=())
def kernel(x, table, W, b):
    pooled = _sc_pool(x, table)
    Wp = jnp.pad(W, ((0, 0), (0, NLAB_PAD - NLAB)))
    bp = jnp.concatenate(
        [b, jnp.full((NLAB_PAD - NLAB,), -1e30, b.dtype)]).reshape(1, NLAB_PAD)
    out = pl.pallas_call(
        _tc_head_body,
        grid=(B // TB,),
        in_specs=[
            pl.BlockSpec((TB, D), lambda i: (i, 0)),
            pl.BlockSpec((D, NLAB_PAD), lambda i: (0, 0)),
            pl.BlockSpec((1, NLAB_PAD), lambda i: (0, 0)),
        ],
        out_specs=pl.BlockSpec((TB, NLAB), lambda i: (i, 0)),
        out_shape=jax.ShapeDtypeStruct((B, NLAB), jnp.float32),
    )(pooled, Wp, bp)
    return out
